# baseline (device time: 38239 ns/iter reference)
import jax
import jax.numpy as jnp
from jax import lax
from jax.experimental import pallas as pl
from jax.experimental.pallas import tpu as pltpu

N_DEV = 4
M = 2048
D = 512
H = 1024
E = 32
E_LOC = E // N_DEV
BLK = M // N_DEV
C = 192


def kernel(x, router_W, route_idx, expert_W, shared_W):
    def body(x_ref, rw_ref, idx_ref, ew_ref, sw_ref, out_ref,
             w_ref, acc_ref, recv_ref, send_sems, recv_sems):
        my = lax.axis_index("i")

        barrier_sem = pltpu.get_barrier_semaphore()
        for j in range(1, N_DEV):
            pl.semaphore_signal(barrier_sem, inc=1,
                                device_id=((my + j) % N_DEV,),
                                device_id_type=pl.DeviceIdType.MESH)
        pl.semaphore_wait(barrier_sem, N_DEV - 1)

        scores = jnp.dot(x_ref[:, :], rw_ref[:, :],
                         preferred_element_type=jnp.float32)
        smax = jnp.max(scores, axis=1, keepdims=True)
        pr = jnp.exp(scores - smax)
        pr = pr / jnp.sum(pr, axis=1, keepdims=True)
        idx = idx_ref[:, :]
        eids = lax.broadcasted_iota(jnp.int32, (M, E), 1)
        sel = jnp.sum(jnp.where(eids == idx, pr, 0.0), axis=1,
                      keepdims=True)
        loc = my * E_LOC + lax.broadcasted_iota(jnp.int32, (1, E_LOC), 1)
        w_ref[:, :] = jnp.where(idx == loc, sel, 0.0)

        r_io = lax.broadcasted_iota(jnp.int32, (BLK, BLK), 0)
        c_io = lax.broadcasted_iota(jnp.int32, (BLK, BLK), 1)
        L = (c_io < r_io).astype(jnp.float32)
        slot_io = lax.broadcasted_iota(jnp.int32, (BLK, C), 1)

        def disp(b, s):
            idx_b = idx_ref[pl.ds(b * BLK, BLK), :]
            mk = (idx_b // E_LOC == s).astype(jnp.float32)
            rank = jnp.dot(L, mk, preferred_element_type=jnp.float32)
            rank_i = rank.astype(jnp.int32)
            return jnp.where((slot_io == rank_i) & (mk > 0.5), 1.0, 0.0)

        tdim = (((0,), (0,)), ((), ()))

        def compact_partial(b, DT):
            xb = x_ref[pl.ds(b * BLK, BLK), :]
            wb = w_ref[pl.ds(b * BLK, BLK), :]
            xc = lax.dot_general(DT, xb, tdim,
                                 preferred_element_type=jnp.float32)
            wc = lax.dot_general(DT, wb, tdim,
                                 preferred_element_type=jnp.float32)
            acc = jnp.zeros((C, H), jnp.float32)
            for le in range(E_LOC):
                acc = acc + jnp.dot(xc * wc[:, le:le + 1], ew_ref[le],
                                    preferred_element_type=jnp.float32)
            return acc

        rdmas = {}
        for j in (2, 1, 3):
            dest = (my + j) % N_DEV
            acc_ref[j - 1] = compact_partial(
                dest, disp(dest, my)).astype(jnp.bfloat16)
            rdma = pltpu.make_async_remote_copy(
                src_ref=acc_ref.at[j - 1],
                dst_ref=recv_ref.at[j - 1],
                send_sem=send_sems.at[j - 1],
                recv_sem=recv_sems.at[j - 1],
                device_id=(dest,),
                device_id_type=pl.DeviceIdType.MESH,
            )
            rdma.start()
            rdmas[j] = rdma

        DT_own = disp(my, my)
        yc_own = compact_partial(my, DT_own)
        xm = x_ref[pl.ds(my * BLK, BLK), :]
        total = jnp.dot(xm, sw_ref[:, :],
                        preferred_element_type=jnp.float32)
        total = total + jnp.dot(DT_own, yc_own,
                                preferred_element_type=jnp.float32)

        DT_in = {j: disp(my, (my + N_DEV - j) % N_DEV) for j in (2, 1, 3)}
        for j in (2, 1, 3):
            rdmas[j].wait_recv()
            total = total + jnp.dot(
                DT_in[j], recv_ref[j - 1].astype(jnp.float32),
                preferred_element_type=jnp.float32)
        out_ref[:, :] = total
        for rdma in rdmas.values():
            rdma.wait_send()

    return pl.pallas_call(
        body,
        out_shape=jax.ShapeDtypeStruct((BLK, H), jnp.float32),
        in_specs=[pl.BlockSpec(memory_space=pltpu.VMEM)] * 5,
        out_specs=pl.BlockSpec(memory_space=pltpu.VMEM),
        scratch_shapes=[
            pltpu.VMEM((M, E_LOC), jnp.float32),
            pltpu.VMEM((N_DEV - 1, C, H), jnp.bfloat16),
            pltpu.VMEM((N_DEV - 1, C, H), jnp.bfloat16),
            pltpu.SemaphoreType.DMA((N_DEV - 1,)),
            pltpu.SemaphoreType.DMA((N_DEV - 1,)),
        ],
        compiler_params=pltpu.CompilerParams(
            collective_id=0,
            vmem_limit_bytes=48 * 1024 * 1024,
        ),
    )(x, router_W, route_idx, expert_W, shared_W)


# device time: 31064 ns/iter; 1.2310x vs baseline; 1.2310x over previous
import jax
import jax.numpy as jnp
from jax import lax
from jax.experimental import pallas as pl
from jax.experimental.pallas import tpu as pltpu

N_DEV = 4
M = 2048
D = 512
H = 1024
E = 32
E_LOC = E // N_DEV
BLK = M // N_DEV
C = 192


def kernel(x, router_W, route_idx, expert_W, shared_W):
    def body(x_ref, rw_ref, idx_ref, ew_ref, sw_ref, out_ref,
             w_ref, xcat_ref, acc_ref, recv_ref, send_sems, recv_sems):
        my = lax.axis_index("i")

        barrier_sem = pltpu.get_barrier_semaphore()
        for j in range(1, N_DEV):
            pl.semaphore_signal(barrier_sem, inc=1,
                                device_id=((my + j) % N_DEV,),
                                device_id_type=pl.DeviceIdType.MESH)
        pl.semaphore_wait(barrier_sem, N_DEV - 1)

        scores = jnp.dot(x_ref[:, :], rw_ref[:, :],
                         preferred_element_type=jnp.float32)
        smax = jnp.max(scores, axis=1, keepdims=True)
        pr = jnp.exp(scores - smax)
        pr = pr / jnp.sum(pr, axis=1, keepdims=True)
        idx = idx_ref[:, :]
        eids = lax.broadcasted_iota(jnp.int32, (M, E), 1)
        sel = jnp.sum(jnp.where(eids == idx, pr, 0.0), axis=1,
                      keepdims=True)
        loc = my * E_LOC + lax.broadcasted_iota(jnp.int32, (1, E_LOC), 1)
        w_ref[:, :] = jnp.where(idx == loc, sel, 0.0)

        r_io = lax.broadcasted_iota(jnp.int32, (BLK, BLK), 0)
        c_io = lax.broadcasted_iota(jnp.int32, (BLK, BLK), 1)
        L = (c_io < r_io).astype(jnp.float32)
        slot_io = lax.broadcasted_iota(jnp.int32, (BLK, C), 1)

        def disp(b, s):
            idx_b = idx_ref[pl.ds(b * BLK, BLK), :]
            mk = (idx_b // E_LOC == s).astype(jnp.float32)
            rank = jnp.dot(L, mk, preferred_element_type=jnp.float32)
            rank_i = rank.astype(jnp.int32)
            return jnp.where((slot_io == rank_i) & (mk > 0.5), 1.0, 0.0)

        tdim = (((0,), (0,)), ((), ()))
        ew_flat = ew_ref[:, :, :].reshape(E_LOC * D, H)

        def compact_partial(b, DT):
            xb = x_ref[pl.ds(b * BLK, BLK), :]
            wb = w_ref[pl.ds(b * BLK, BLK), :]
            xc = lax.dot_general(DT, xb, tdim,
                                 preferred_element_type=jnp.float32)
            wc = lax.dot_general(DT, wb, tdim,
                                 preferred_element_type=jnp.float32)
            for le in range(E_LOC):
                xcat_ref[:, le * D:(le + 1) * D] = xc * wc[:, le:le + 1]
            return jnp.dot(xcat_ref[:, :], ew_flat,
                           preferred_element_type=jnp.float32)

        rdmas = {}
        for j in (2, 1, 3):
            dest = (my + j) % N_DEV
            acc_ref[j - 1] = compact_partial(
                dest, disp(dest, my)).astype(jnp.bfloat16)
            rdma = pltpu.make_async_remote_copy(
                src_ref=acc_ref.at[j - 1],
                dst_ref=recv_ref.at[j - 1],
                send_sem=send_sems.at[j - 1],
                recv_sem=recv_sems.at[j - 1],
                device_id=(dest,),
                device_id_type=pl.DeviceIdType.MESH,
            )
            rdma.start()
            rdmas[j] = rdma

        DT_own = disp(my, my)
        yc_own = compact_partial(my, DT_own)
        xm = x_ref[pl.ds(my * BLK, BLK), :]
        total = jnp.dot(xm, sw_ref[:, :],
                        preferred_element_type=jnp.float32)
        total = total + jnp.dot(DT_own, yc_own,
                                preferred_element_type=jnp.float32)

        DT_in = {j: disp(my, (my + N_DEV - j) % N_DEV) for j in (2, 1, 3)}
        for j in (2, 1, 3):
            rdmas[j].wait_recv()
            total = total + jnp.dot(
                DT_in[j], recv_ref[j - 1].astype(jnp.float32),
                preferred_element_type=jnp.float32)
        out_ref[:, :] = total
        for rdma in rdmas.values():
            rdma.wait_send()

    return pl.pallas_call(
        body,
        out_shape=jax.ShapeDtypeStruct((BLK, H), jnp.float32),
        in_specs=[pl.BlockSpec(memory_space=pltpu.VMEM)] * 5,
        out_specs=pl.BlockSpec(memory_space=pltpu.VMEM),
        scratch_shapes=[
            pltpu.VMEM((M, E_LOC), jnp.float32),
            pltpu.VMEM((C, E_LOC * D), jnp.float32),
            pltpu.VMEM((N_DEV - 1, C, H), jnp.bfloat16),
            pltpu.VMEM((N_DEV - 1, C, H), jnp.bfloat16),
            pltpu.SemaphoreType.DMA((N_DEV - 1,)),
            pltpu.SemaphoreType.DMA((N_DEV - 1,)),
        ],
        compiler_params=pltpu.CompilerParams(
            collective_id=0,
            vmem_limit_bytes=48 * 1024 * 1024,
        ),
    )(x, router_W, route_idx, expert_W, shared_W)
